# baseline (speedup 1.0000x reference)
import functools

import jax
import jax.numpy as jnp
from jax import lax
from jax.experimental import pallas as pl
from jax.experimental.pallas import tpu as pltpu
from jax.experimental.pallas import tpu_sc as plsc

N_NODES = 10000
N_EDGES = 320000
IN_DIM = 128
HID_DIM = 128
OUT_DIM = 64

NP = 10240
NC = 2
NS = 16
NW = NC * NS
CHUNK = 128
NCH = 79
EPW = NCH * CHUNK
EP = NW * EPW
STRIPE = NP // NS


def _worker_ids():
    c = lax.axis_index("c")
    s = lax.axis_index("s")
    return c, s, c * NS + s


def _fill_vmem_1d(ref, n, value):
    def body(i, carry):
        ref[pl.ds(i * 16, 16)] = jnp.full((16,), value, jnp.float32)
        return carry
    lax.fori_loop(0, n // 16, body, 0)


def _fill_vmem_2d(ref, rows, cols, value):
    per_row = cols // 16
    def body(i, carry):
        r = i // per_row
        col = (i % per_row) * 16
        ref[r, pl.ds(col, 16)] = jnp.full((16,), value, jnp.float32)
        return carry
    lax.fori_loop(0, rows * per_row, body, 0)


@functools.partial(
    pl.kernel,
    out_type=jax.ShapeDtypeStruct((NC, NP), jnp.float32),
    mesh=plsc.VectorSubcoreMesh(core_axis_name="c", subcore_axis_name="s"),
    scratch_types=[
        pltpu.VMEM((NCH, CHUNK), jnp.int32),
        pltpu.VMEM((CHUNK,), jnp.float32),
        pltpu.VMEM((STRIPE,), jnp.float32),
        pltpu.VMEM_SHARED((NP,), jnp.float32),
    ],
)
def _degree_kernel(dst_hbm, deg_out, idx_v, ones_v, zeros_v, deg_sh):
    c, s, w = _worker_ids()
    _fill_vmem_1d(zeros_v, STRIPE, 0.0)
    _fill_vmem_1d(ones_v, CHUNK, 1.0)
    pltpu.sync_copy(zeros_v, deg_sh.at[pl.ds(s * STRIPE, STRIPE)])
    plsc.subcore_barrier()

    pltpu.sync_copy(dst_hbm.at[w], idx_v)

    def body(ch, carry):
        pltpu.sync_copy(ones_v, deg_sh.at[idx_v.at[ch]], add=True)
        return carry
    lax.fori_loop(0, NCH, body, 0)

    plsc.subcore_barrier()
    pltpu.sync_copy(deg_sh.at[pl.ds(s * STRIPE, STRIPE)],
                    deg_out.at[c, pl.ds(s * STRIPE, STRIPE)])


def _make_scatter_kernel(d):

    @functools.partial(
        pl.kernel,
        out_type=jax.ShapeDtypeStruct((NC, NP, d), jnp.float32),
        mesh=plsc.VectorSubcoreMesh(core_axis_name="c", subcore_axis_name="s"),
        scratch_types=[
            pltpu.VMEM((NCH, CHUNK), jnp.int32),
            pltpu.VMEM((2, CHUNK), jnp.int32),
            pltpu.VMEM((2, CHUNK, d), jnp.float32),
            pltpu.VMEM_SHARED((NP, d), jnp.float32),
            pltpu.SemaphoreType.DMA,
            pltpu.SemaphoreType.DMA,
            pltpu.SemaphoreType.DMA,
        ],
    )
    def scatter_kernel(hs_hbm, src_hbm, dst_hbm, out_hbm,
                       src_v, dst_v, rows_v, acc_sh, gsem, isem, ssem):
        c, s, w = _worker_ids()
        _fill_vmem_2d(rows_v.at[0], CHUNK, d, 0.0)
        for j in range(STRIPE // CHUNK):
            pltpu.sync_copy(
                rows_v.at[0], acc_sh.at[pl.ds(s * STRIPE + j * CHUNK, CHUNK)])
        plsc.subcore_barrier()

        pltpu.sync_copy(src_hbm.at[w], src_v)

        pltpu.async_copy(hs_hbm.at[src_v.at[0]], rows_v.at[0], gsem)
        pltpu.async_copy(dst_hbm.at[w, 0], dst_v.at[0], isem)

        def body(ch, carry):
            b = ch % 2
            pltpu.make_async_copy(
                hs_hbm.at[src_v.at[ch]], rows_v.at[b], gsem).wait()
            pltpu.make_async_copy(
                dst_hbm.at[w, ch], dst_v.at[b], isem).wait()

            @pl.when(ch >= 1)
            def _free_bufs():
                pltpu.make_async_copy(
                    rows_v.at[b], acc_sh.at[dst_v.at[b]], ssem).wait()

            @pl.when(ch + 1 < NCH)
            def _prefetch():
                pltpu.async_copy(
                    hs_hbm.at[src_v.at[ch + 1]], rows_v.at[1 - b], gsem)
                pltpu.async_copy(dst_hbm.at[w, ch + 1], dst_v.at[1 - b], isem)

            pltpu.async_copy(
                rows_v.at[b], acc_sh.at[dst_v.at[b]], ssem, add=True)
            return carry
        lax.fori_loop(0, NCH, body, 0)

        pltpu.make_async_copy(
            rows_v.at[0], acc_sh.at[dst_v.at[0]], ssem).wait()

        plsc.subcore_barrier()
        pltpu.sync_copy(acc_sh.at[pl.ds(s * STRIPE, STRIPE)],
                        out_hbm.at[c, pl.ds(s * STRIPE, STRIPE)])

    return scatter_kernel


_scatter128 = _make_scatter_kernel(HID_DIM)


BR = 1024


def _dinv_of(deg_ref):
    deg = deg_ref[0, :] + deg_ref[1, :] + 1.0
    return lax.rsqrt(deg)


def _k1_body(x_ref, w_ref, deg_ref, o_ref):
    h = jnp.dot(x_ref[...], w_ref[...], preferred_element_type=jnp.float32)
    o_ref[...] = h * _dinv_of(deg_ref)[:, None]


def _k2_body(agg_ref, hs_ref, w_ref, deg_ref, b_ref, o_ref):
    dinv = _dinv_of(deg_ref)[:, None]
    tmp = agg_ref[0] + agg_ref[1] + hs_ref[...]
    u = jnp.maximum(tmp * dinv + b_ref[...], 0.0)
    h2 = jnp.dot(u, w_ref[...], preferred_element_type=jnp.float32)
    o_ref[...] = h2 * dinv


def _k3_body(agg_ref, hs_ref, deg_ref, b_ref, o_ref):
    dinv = _dinv_of(deg_ref)[:, None]
    s = (agg_ref[0] + agg_ref[1] + hs_ref[...]) * dinv
    o_ref[...] = s[:, :OUT_DIM] + b_ref[...]


def _tc_k1(x_p, w1, deg_parts):
    return pl.pallas_call(
        _k1_body,
        grid=(NP // BR,),
        in_specs=[
            pl.BlockSpec((BR, IN_DIM), lambda i: (i, 0)),
            pl.BlockSpec((IN_DIM, HID_DIM), lambda i: (0, 0)),
            pl.BlockSpec((NC, BR), lambda i: (0, i)),
        ],
        out_specs=pl.BlockSpec((BR, HID_DIM), lambda i: (i, 0)),
        out_shape=jax.ShapeDtypeStruct((NP, HID_DIM), jnp.float32),
    )(x_p, w1, deg_parts)


def _tc_k2(agg1, hs1, w2, deg_parts, b1):
    return pl.pallas_call(
        _k2_body,
        grid=(NP // BR,),
        in_specs=[
            pl.BlockSpec((NC, BR, HID_DIM), lambda i: (0, i, 0)),
            pl.BlockSpec((BR, HID_DIM), lambda i: (i, 0)),
            pl.BlockSpec((HID_DIM, HID_DIM), lambda i: (0, 0)),
            pl.BlockSpec((NC, BR), lambda i: (0, i)),
            pl.BlockSpec((1, HID_DIM), lambda i: (0, 0)),
        ],
        out_specs=pl.BlockSpec((BR, HID_DIM), lambda i: (i, 0)),
        out_shape=jax.ShapeDtypeStruct((NP, HID_DIM), jnp.float32),
    )(agg1, hs1, w2, deg_parts, b1)


def _tc_k3(agg2, hs2, deg_parts, b2):
    return pl.pallas_call(
        _k3_body,
        grid=(NP // BR,),
        in_specs=[
            pl.BlockSpec((NC, BR, HID_DIM), lambda i: (0, i, 0)),
            pl.BlockSpec((BR, HID_DIM), lambda i: (i, 0)),
            pl.BlockSpec((NC, BR), lambda i: (0, i)),
            pl.BlockSpec((1, OUT_DIM), lambda i: (0, 0)),
        ],
        out_specs=pl.BlockSpec((BR, OUT_DIM), lambda i: (i, 0)),
        out_shape=jax.ShapeDtypeStruct((NP, OUT_DIM), jnp.float32),
    )(agg2, hs2, deg_parts, b2)


def kernel(x, edge_index, W1, b1, W2, b2):
    src = edge_index[0].astype(jnp.int32)
    dst = edge_index[1].astype(jnp.int32)

    npad = EP - N_EDGES
    pad_idx = N_NODES + (jnp.arange(npad, dtype=jnp.int32) % (NP - N_NODES))
    src_p = jnp.concatenate([src, pad_idx]).reshape(NW, NCH, CHUNK)
    dst_p = jnp.concatenate([dst, pad_idx]).reshape(NW, NCH, CHUNK)

    x_p = jnp.pad(x, ((0, NP - N_NODES), (0, 0)))
    w2_p = jnp.pad(W2, ((0, 0), (0, HID_DIM - OUT_DIM)))
    b1r = b1.reshape(1, HID_DIM)
    b2r = b2.reshape(1, OUT_DIM)

    deg_parts = _degree_kernel(dst_p)
    hs1 = _tc_k1(x_p, W1, deg_parts)
    agg1 = _scatter128(hs1, src_p, dst_p)
    hs2 = _tc_k2(agg1, hs1, w2_p, deg_parts, b1r)
    agg2 = _scatter128(hs2, src_p, dst_p)
    out = _tc_k3(agg2, hs2, deg_parts, b2r)
    return out[:N_NODES]


# speedup vs baseline: 1.0824x; 1.0824x over previous
import functools

import jax
import jax.numpy as jnp
from jax import lax
from jax.experimental import pallas as pl
from jax.experimental.pallas import tpu as pltpu
from jax.experimental.pallas import tpu_sc as plsc

N_NODES = 10000
N_EDGES = 320000
IN_DIM = 128
HID_DIM = 128
OUT_DIM = 64

NP = 10240
NC = 2
NS = 16
NW = NC * NS
CHUNK = 128
NCH = 79
EPW = NCH * CHUNK
EP = NW * EPW
STRIPE = NP // NS


def _worker_ids():
    c = lax.axis_index("c")
    s = lax.axis_index("s")
    return c, s, c * NS + s


def _fill_vmem_1d(ref, n, value):
    def body(i, carry):
        ref[pl.ds(i * 16, 16)] = jnp.full((16,), value, jnp.float32)
        return carry
    lax.fori_loop(0, n // 16, body, 0)


def _fill_vmem_2d(ref, rows, cols, value):
    per_row = cols // 16
    def body(i, carry):
        r = i // per_row
        col = (i % per_row) * 16
        ref[r, pl.ds(col, 16)] = jnp.full((16,), value, jnp.float32)
        return carry
    lax.fori_loop(0, rows * per_row, body, 0)


@functools.partial(
    pl.kernel,
    out_type=jax.ShapeDtypeStruct((NC, NP), jnp.float32),
    mesh=plsc.VectorSubcoreMesh(core_axis_name="c", subcore_axis_name="s"),
    scratch_types=[
        pltpu.VMEM((NCH, CHUNK), jnp.int32),
        pltpu.VMEM((CHUNK,), jnp.float32),
        pltpu.VMEM((STRIPE,), jnp.float32),
        pltpu.VMEM_SHARED((NP,), jnp.float32),
    ],
)
def _degree_kernel(dst_hbm, deg_out, idx_v, ones_v, zeros_v, deg_sh):
    c, s, w = _worker_ids()
    _fill_vmem_1d(zeros_v, STRIPE, 0.0)
    _fill_vmem_1d(ones_v, CHUNK, 1.0)
    pltpu.sync_copy(zeros_v, deg_sh.at[pl.ds(s * STRIPE, STRIPE)])
    plsc.subcore_barrier()

    pltpu.sync_copy(dst_hbm.at[w], idx_v)

    def body(ch, carry):
        pltpu.sync_copy(ones_v, deg_sh.at[idx_v.at[ch]], add=True)
        return carry
    lax.fori_loop(0, NCH, body, 0)

    plsc.subcore_barrier()
    pltpu.sync_copy(deg_sh.at[pl.ds(s * STRIPE, STRIPE)],
                    deg_out.at[c, pl.ds(s * STRIPE, STRIPE)])


def _make_scatter_kernel(d, tc_tiling=True):

    @functools.partial(
        pl.kernel,
        out_type=jax.ShapeDtypeStruct((NC, NP, d), jnp.float32),
        mesh=plsc.VectorSubcoreMesh(core_axis_name="c", subcore_axis_name="s"),
        compiler_params=pltpu.CompilerParams(use_tc_tiling_on_sc=tc_tiling),
        scratch_types=[
            pltpu.VMEM((NCH, CHUNK), jnp.int32),
            pltpu.VMEM((2, CHUNK), jnp.int32),
            pltpu.VMEM((2, CHUNK, d), jnp.float32),
            pltpu.VMEM_SHARED((NP, d), jnp.float32),
            pltpu.SemaphoreType.DMA,
            pltpu.SemaphoreType.DMA,
            pltpu.SemaphoreType.DMA,
        ],
    )
    def scatter_kernel(hs_hbm, src_hbm, dst_hbm, out_hbm,
                       src_v, dst_v, rows_v, acc_sh, gsem, isem, ssem):
        c, s, w = _worker_ids()
        _fill_vmem_2d(rows_v.at[0], CHUNK, d, 0.0)
        for j in range(STRIPE // CHUNK):
            pltpu.sync_copy(
                rows_v.at[0], acc_sh.at[pl.ds(s * STRIPE + j * CHUNK, CHUNK)])
        plsc.subcore_barrier()

        pltpu.sync_copy(src_hbm.at[w], src_v)

        pltpu.async_copy(hs_hbm.at[src_v.at[0]], rows_v.at[0], gsem)
        pltpu.async_copy(dst_hbm.at[w, 0], dst_v.at[0], isem)

        def body(ch, carry):
            b = ch % 2
            pltpu.make_async_copy(
                hs_hbm.at[src_v.at[ch]], rows_v.at[b], gsem).wait()
            pltpu.make_async_copy(
                dst_hbm.at[w, ch], dst_v.at[b], isem).wait()

            @pl.when(ch >= 1)
            def _free_bufs():
                pltpu.make_async_copy(
                    rows_v.at[b], acc_sh.at[dst_v.at[b]], ssem).wait()

            @pl.when(ch + 1 < NCH)
            def _prefetch():
                pltpu.async_copy(
                    hs_hbm.at[src_v.at[ch + 1]], rows_v.at[1 - b], gsem)
                pltpu.async_copy(dst_hbm.at[w, ch + 1], dst_v.at[1 - b], isem)

            pltpu.async_copy(
                rows_v.at[b], acc_sh.at[dst_v.at[b]], ssem, add=True)
            return carry
        lax.fori_loop(0, NCH, body, 0)

        pltpu.make_async_copy(
            rows_v.at[0], acc_sh.at[dst_v.at[0]], ssem).wait()

        plsc.subcore_barrier()
        pltpu.sync_copy(acc_sh.at[pl.ds(s * STRIPE, STRIPE)],
                        out_hbm.at[c, pl.ds(s * STRIPE, STRIPE)])

    return scatter_kernel


_scatter128 = _make_scatter_kernel(HID_DIM)


_scatter64 = _make_scatter_kernel(OUT_DIM, tc_tiling=False)


BR = 1024


def _dinv_of(deg_ref):
    deg = deg_ref[0, :] + deg_ref[1, :] + 1.0
    return lax.rsqrt(deg)


def _k1_body(x_ref, w_ref, deg_ref, o_ref):
    h = jnp.dot(x_ref[...], w_ref[...], preferred_element_type=jnp.float32)
    o_ref[...] = h * _dinv_of(deg_ref)[:, None]


def _k2_body(agg_ref, hs_ref, w_ref, deg_ref, b_ref, o_ref):
    dinv = _dinv_of(deg_ref)[:, None]
    tmp = agg_ref[0] + agg_ref[1] + hs_ref[...]
    u = jnp.maximum(tmp * dinv + b_ref[...], 0.0)
    h2 = jnp.dot(u, w_ref[...], preferred_element_type=jnp.float32)
    o_ref[...] = h2 * dinv


def _k3_body(agg_ref, hs_ref, deg_ref, b_ref, o_ref):
    dinv = _dinv_of(deg_ref)[:, None]
    o_ref[...] = (agg_ref[0] + agg_ref[1] + hs_ref[...]) * dinv + b_ref[...]


def _tc_k1(x_p, w1, deg_parts):
    return pl.pallas_call(
        _k1_body,
        grid=(NP // BR,),
        in_specs=[
            pl.BlockSpec((BR, IN_DIM), lambda i: (i, 0)),
            pl.BlockSpec((IN_DIM, HID_DIM), lambda i: (0, 0)),
            pl.BlockSpec((NC, BR), lambda i: (0, i)),
        ],
        out_specs=pl.BlockSpec((BR, HID_DIM), lambda i: (i, 0)),
        out_shape=jax.ShapeDtypeStruct((NP, HID_DIM), jnp.float32),
    )(x_p, w1, deg_parts)


def _tc_k2(agg1, hs1, w2, deg_parts, b1):
    return pl.pallas_call(
        _k2_body,
        grid=(NP // BR,),
        in_specs=[
            pl.BlockSpec((NC, BR, HID_DIM), lambda i: (0, i, 0)),
            pl.BlockSpec((BR, HID_DIM), lambda i: (i, 0)),
            pl.BlockSpec((HID_DIM, OUT_DIM), lambda i: (0, 0)),
            pl.BlockSpec((NC, BR), lambda i: (0, i)),
            pl.BlockSpec((1, HID_DIM), lambda i: (0, 0)),
        ],
        out_specs=pl.BlockSpec((BR, OUT_DIM), lambda i: (i, 0)),
        out_shape=jax.ShapeDtypeStruct((NP, OUT_DIM), jnp.float32),
    )(agg1, hs1, w2, deg_parts, b1)


def _tc_k3(agg2, hs2, deg_parts, b2):
    return pl.pallas_call(
        _k3_body,
        grid=(NP // BR,),
        in_specs=[
            pl.BlockSpec((NC, BR, OUT_DIM), lambda i: (0, i, 0)),
            pl.BlockSpec((BR, OUT_DIM), lambda i: (i, 0)),
            pl.BlockSpec((NC, BR), lambda i: (0, i)),
            pl.BlockSpec((1, OUT_DIM), lambda i: (0, 0)),
        ],
        out_specs=pl.BlockSpec((BR, OUT_DIM), lambda i: (i, 0)),
        out_shape=jax.ShapeDtypeStruct((NP, OUT_DIM), jnp.float32),
    )(agg2, hs2, deg_parts, b2)


def kernel(x, edge_index, W1, b1, W2, b2):
    src = edge_index[0].astype(jnp.int32)
    dst = edge_index[1].astype(jnp.int32)

    npad = EP - N_EDGES
    pad_idx = N_NODES + (jnp.arange(npad, dtype=jnp.int32) % (NP - N_NODES))
    src_p = jnp.concatenate([src, pad_idx]).reshape(NW, NCH, CHUNK)
    dst_p = jnp.concatenate([dst, pad_idx]).reshape(NW, NCH, CHUNK)

    x_p = jnp.pad(x, ((0, NP - N_NODES), (0, 0)))
    b1r = b1.reshape(1, HID_DIM)
    b2r = b2.reshape(1, OUT_DIM)

    deg_parts = _degree_kernel(dst_p)
    hs1 = _tc_k1(x_p, W1, deg_parts)
    agg1 = _scatter128(hs1, src_p, dst_p)
    hs2 = _tc_k2(agg1, hs1, W2, deg_parts, b1r)
    agg2 = _scatter64(hs2, src_p, dst_p)
    out = _tc_k3(agg2, hs2, deg_parts, b2r)
    return out[:N_NODES]
